# baseline (speedup 1.0000x reference)
import jax
import jax.numpy as jnp
from jax import lax
from jax.experimental import pallas as pl
from jax.experimental.pallas import tpu as pltpu
from jax.experimental.pallas import tpu_sc as plsc

BATCH = 1024
SEQ = 200
EMBED = 128
HALF = 100
NC = 2
NS = 16
NW = NC * NS
SEQ_PER_W = BATCH // NW
NV = EMBED // 16
NBUF = 3


def _emb_body(idx_hbm, tok_hbm, pos_hbm, out_hbm, idx_v, rows_v, pos_v,
              gsem0, gsem1, gsem2, ssem0, ssem1, ssem2):
    wid = lax.axis_index("s") * NC + lax.axis_index("c")
    base_seq = wid * SEQ_PER_W
    gsems = (gsem0, gsem1, gsem2)
    ssems = (ssem0, ssem1, ssem2)

    pltpu.sync_copy(idx_hbm.at[pl.ds(base_seq, SEQ_PER_W)], idx_v)
    pltpu.sync_copy(pos_hbm, pos_v)

    def gather_descs(i, b, issue):
        mk = pltpu.async_copy if issue else pltpu.make_async_copy
        c0 = mk(tok_hbm.at[idx_v.at[i, 0]], rows_v.at[b, pl.ds(0, HALF)],
                gsems[b])
        c1 = mk(tok_hbm.at[idx_v.at[i, 1]], rows_v.at[b, pl.ds(HALF, HALF)],
                gsems[b])
        return c0, c1

    def wait_gather(i, b):
        for c in gather_descs(i, b, issue=False):
            c.wait()

    def start_store(i, b):
        pltpu.async_copy(rows_v.at[b], out_hbm.at[base_seq + i], ssems[b])

    def wait_store(i, b):
        pltpu.make_async_copy(rows_v.at[b], out_hbm.at[base_seq + i],
                              ssems[b]).wait()

    def add_pos(b):
        def body(r, _):
            for u in range(2):
                rr = r * 2 + u
                for j in range(NV):
                    sl = pl.ds(j * 16, 16)
                    rows_v[b, rr, sl] = rows_v[b, rr, sl] + pos_v[rr, sl]
            return ()
        lax.fori_loop(0, SEQ // 2, body, ())

    def step(i, b, first=False):
        if not first:
            wait_store(i - 1, (b + NBUF - 1) % NBUF)
        gather_descs(i + 2, (b + 2) % NBUF, issue=True)
        wait_gather(i, b)
        add_pos(b)
        start_store(i, b)

    gather_descs(0, 0, issue=True)
    gather_descs(1, 1, issue=True)
    step(0, 0, first=True)
    step(1, 1)

    def outer(o, _):
        for u in range(NBUF):
            i = 2 + o * NBUF + u
            b = (2 + u) % NBUF

            wait_store(i - 1, (b + NBUF - 1) % NBUF)

            @pl.when(i + 2 < SEQ_PER_W)
            def _():
                gather_descs(i + 2, (b + 2) % NBUF, issue=True)

            wait_gather(i, b)
            add_pos(b)
            start_store(i, b)
        return ()

    lax.fori_loop(0, (SEQ_PER_W - 2) // NBUF, outer, ())
    wait_store(SEQ_PER_W - 1, (SEQ_PER_W - 1) % NBUF)


@jax.jit
def kernel(inputs, token_table, pos_table):
    idx = inputs.reshape(BATCH, 2, HALF).astype(jnp.int32)
    mesh = plsc.VectorSubcoreMesh(core_axis_name="c", subcore_axis_name="s")
    run = pl.kernel(
        _emb_body,
        out_type=jax.ShapeDtypeStruct((BATCH, SEQ, EMBED), jnp.float32),
        mesh=mesh,
        scratch_types=[
            pltpu.VMEM((SEQ_PER_W, 2, HALF), jnp.int32),
            pltpu.VMEM((NBUF, SEQ, EMBED), jnp.float32),
            pltpu.VMEM((SEQ, EMBED), jnp.float32),
            pltpu.SemaphoreType.DMA,
            pltpu.SemaphoreType.DMA,
            pltpu.SemaphoreType.DMA,
            pltpu.SemaphoreType.DMA,
            pltpu.SemaphoreType.DMA,
            pltpu.SemaphoreType.DMA,
        ],
    )
    return run(idx, token_table, pos_table)


# speedup vs baseline: 1.1693x; 1.1693x over previous
import jax
import jax.numpy as jnp
from jax import lax
from jax.experimental import pallas as pl
from jax.experimental.pallas import tpu as pltpu
from jax.experimental.pallas import tpu_sc as plsc

BATCH = 1024
SEQ = 200
EMBED = 128
HALF = 100
NC = 2
NS = 16
NW = NC * NS
SEQ_PER_W = BATCH // NW
NV = EMBED // 16
NBUF = 2


def _emb_body(idx_hbm, tok_hbm, pos_hbm, out_hbm,
              idx_v, rows_v, pos_v, gsem0, gsem1, ssem0, ssem1):
    wid = lax.axis_index("s") * NC + lax.axis_index("c")
    base_seq = wid * SEQ_PER_W
    gsems = (gsem0, gsem1)
    ssems = (ssem0, ssem1)

    pltpu.sync_copy(idx_hbm.at[pl.ds(base_seq, SEQ_PER_W)], idx_v)
    pltpu.sync_copy(pos_hbm, pos_v)

    def gather_descs(i, b, issue):
        mk = pltpu.async_copy if issue else pltpu.make_async_copy
        c0 = mk(tok_hbm.at[idx_v.at[i, 0]], rows_v.at[b, pl.ds(0, HALF)],
                gsems[b])
        c1 = mk(tok_hbm.at[idx_v.at[i, 1]], rows_v.at[b, pl.ds(HALF, HALF)],
                gsems[b])
        return c0, c1

    def wait_gather(i, b):
        for c in gather_descs(i, b, issue=False):
            c.wait()

    NQ = 5
    QROWS = SEQ // NQ

    def start_store_q(i, b, q):
        sl = pl.ds(q * QROWS, QROWS)
        pltpu.async_copy(rows_v.at[b, sl], out_hbm.at[base_seq + i, sl],
                         ssems[b])

    def wait_store(i, b):
        pltpu.make_async_copy(rows_v.at[b], out_hbm.at[base_seq + i],
                              ssems[b]).wait()

    def add_pos_q(b, q):
        def body(r, _):
            for u in range(2):
                rr = q * QROWS + r * 2 + u
                for j in range(NV):
                    sl = pl.ds(j * 16, 16)
                    rows_v[b, rr, sl] = rows_v[b, rr, sl] + pos_v[rr, sl]
            return ()
        lax.fori_loop(0, QROWS // 2, body, ())

    gather_descs(0, 0, issue=True)

    def outer(o, _):
        for b in range(NBUF):
            i = o * NBUF + b
            bn = 1 - b

            @pl.when(i + 1 < SEQ_PER_W)
            def _():
                @pl.when(i >= 1)
                def _():
                    wait_store(i - 1, bn)
                gather_descs(i + 1, bn, issue=True)

            wait_gather(i, b)
            for q in range(NQ):
                add_pos_q(b, q)
                start_store_q(i, b, q)
        return ()

    lax.fori_loop(0, SEQ_PER_W // NBUF, outer, ())
    wait_store(SEQ_PER_W - 2, 0)
    wait_store(SEQ_PER_W - 1, 1)


@jax.jit
def kernel(inputs, token_table, pos_table):
    idx = inputs.reshape(BATCH, 2, HALF).astype(jnp.int32)
    mesh = plsc.VectorSubcoreMesh(core_axis_name="c", subcore_axis_name="s")
    run = pl.kernel(
        _emb_body,
        out_type=jax.ShapeDtypeStruct((BATCH, SEQ, EMBED), jnp.float32),
        mesh=mesh,
        scratch_types=[
            pltpu.VMEM((SEQ_PER_W, 2, HALF), jnp.int32),
            pltpu.VMEM((NBUF, SEQ, EMBED), jnp.float32),
            pltpu.VMEM((SEQ, EMBED), jnp.float32),
            pltpu.SemaphoreType.DMA,
            pltpu.SemaphoreType.DMA,
            pltpu.SemaphoreType.DMA,
            pltpu.SemaphoreType.DMA,
        ],
    )
    return run(idx, token_table, pos_table)


# speedup vs baseline: 1.1739x; 1.0039x over previous
import jax
import jax.numpy as jnp
from jax import lax
from jax.experimental import pallas as pl
from jax.experimental.pallas import tpu as pltpu
from jax.experimental.pallas import tpu_sc as plsc

BATCH = 1024
SEQ = 200
EMBED = 128
HALF = 100
NC = 2
NS = 16
NW = NC * NS
SEQ_PER_W = BATCH // NW
NV = EMBED // 16
NBUF = 2


def _emb_body(idx_hbm, tok_hbm, pos_hbm, out_hbm, idx_v, rows_v, pos_v,
              gsem00, gsem01, gsem10, gsem11, ssem0, ssem1):
    wid = lax.axis_index("s") * NC + lax.axis_index("c")
    base_seq = wid * SEQ_PER_W
    gsems = ((gsem00, gsem01), (gsem10, gsem11))
    ssems = (ssem0, ssem1)

    pltpu.sync_copy(idx_hbm.at[pl.ds(base_seq, SEQ_PER_W)], idx_v)
    pltpu.sync_copy(pos_hbm, pos_v)

    def gather_desc(i, b, h, issue):
        mk = pltpu.async_copy if issue else pltpu.make_async_copy
        c = mk(tok_hbm.at[idx_v.at[i, h]], rows_v.at[b, pl.ds(h * HALF, HALF)],
               gsems[b][h])
        if not issue:
            c.wait()

    def issue_gathers(i, b):
        gather_desc(i, b, 0, issue=True)
        gather_desc(i, b, 1, issue=True)

    NQ = 5
    QROWS = SEQ // NQ

    def start_store_q(i, b, q):
        sl = pl.ds(q * QROWS, QROWS)
        pltpu.async_copy(rows_v.at[b, sl], out_hbm.at[base_seq + i, sl],
                         ssems[b])

    def wait_store(i, b):
        pltpu.make_async_copy(rows_v.at[b], out_hbm.at[base_seq + i],
                              ssems[b]).wait()

    def add_pos_q(b, q):
        def body(r, _):
            for u in range(4):
                rr = q * QROWS + r * 4 + u
                for j in range(NV):
                    sl = pl.ds(j * 16, 16)
                    rows_v[b, rr, sl] = rows_v[b, rr, sl] + pos_v[rr, sl]
            return ()
        lax.fori_loop(0, QROWS // 4, body, ())

    issue_gathers(0, 0)

    def outer(o, _):
        for b in range(NBUF):
            i = o * NBUF + b
            bn = 1 - b

            @pl.when(i + 1 < SEQ_PER_W)
            def _():
                @pl.when(i >= 1)
                def _():
                    wait_store(i - 1, bn)
                issue_gathers(i + 1, bn)

            gather_desc(i, b, 0, issue=False)
            for q in range(NQ):
                if q * QROWS <= HALF < (q + 1) * QROWS:
                    gather_desc(i, b, 1, issue=False)
                add_pos_q(b, q)
                start_store_q(i, b, q)
        return ()

    lax.fori_loop(0, SEQ_PER_W // NBUF, outer, ())
    wait_store(SEQ_PER_W - 2, 0)
    wait_store(SEQ_PER_W - 1, 1)


@jax.jit
def kernel(inputs, token_table, pos_table):
    idx = inputs.reshape(BATCH, 2, HALF).astype(jnp.int32)
    mesh = plsc.VectorSubcoreMesh(core_axis_name="c", subcore_axis_name="s")
    run = pl.kernel(
        _emb_body,
        out_type=jax.ShapeDtypeStruct((BATCH, SEQ, EMBED), jnp.float32),
        mesh=mesh,
        scratch_types=[
            pltpu.VMEM((SEQ_PER_W, 2, HALF), jnp.int32),
            pltpu.VMEM((NBUF, SEQ, EMBED), jnp.float32),
            pltpu.VMEM((SEQ, EMBED), jnp.float32),
            pltpu.SemaphoreType.DMA,
            pltpu.SemaphoreType.DMA,
            pltpu.SemaphoreType.DMA,
            pltpu.SemaphoreType.DMA,
            pltpu.SemaphoreType.DMA,
            pltpu.SemaphoreType.DMA,
        ],
    )
    return run(idx, token_table, pos_table)
